# Initial kernel scaffold; baseline (speedup 1.0000x reference)
import jax
import jax.numpy as jnp
from jax.experimental import pallas as pl


def kernel(input):
    raise NotImplementedError("write your pallas kernel here")


# speedup vs baseline: 10.4199x; 10.4199x over previous
import functools

import jax
import jax.numpy as jnp
from jax import lax
from jax.experimental import pallas as pl
from jax.experimental.pallas import tpu as pltpu
from jax.experimental.pallas import tpu_sc as plsc

N = 1048576
C = 16
K = 104858
NC = 2
NS = 16
NW = NC * NS
ROWS_PER = N // NW
CH = 1024
NCH = ROWS_PER // CH

NB1 = 2048
NB2 = 2048
NB3 = 1024


def _make_pass(level, nb):
    mesh = plsc.VectorSubcoreMesh(core_axis_name="c", subcore_axis_name="s")
    out_type = [
        jax.ShapeDtypeStruct((NW, nb * C), jnp.float32),
        jax.ShapeDtypeStruct((NW, nb * C), jnp.float32),
    ]
    scratch = [
        pltpu.VMEM((CH * C,), jnp.float32),
        pltpu.VMEM((CH * C,), jnp.float32),
        pltpu.VMEM((nb * C,), jnp.float32),
        pltpu.VMEM((nb * C,), jnp.float32),
        pltpu.VMEM((C,), jnp.int32),
        pltpu.SemaphoreType.DMA,
        pltpu.SemaphoreType.DMA,
    ]

    def body(x_hbm, p_hbm, cnt_hbm, sum_hbm, buf0, buf1, hcnt, hsum, pv,
             sem0, sem1):
        cid = lax.axis_index("c")
        sid = lax.axis_index("s")
        wid = sid * NC + cid
        base = wid * ROWS_PER * C

        zero = jnp.zeros((C,), jnp.float32)

        def zb(i, carry):
            hcnt[pl.ds(i * C, C)] = zero
            hsum[pl.ds(i * C, C)] = zero
            return carry

        lax.fori_loop(0, nb, zb, 0)

        pltpu.sync_copy(p_hbm, pv)
        pvec = pv[...]
        lanes = lax.iota(jnp.int32, C)
        ones = jnp.full((C,), 1.0, jnp.float32)

        def do_rows(buf):
            def rb(i, carry):
                v = buf[pl.ds(i * C, C)]
                u = lax.bitcast_convert_type(v, jnp.int32)
                skey = jnp.where(u < 0, u ^ 0x7FFFFFFF, u)
                if level == 1:
                    b = (skey >> 21) + 1024
                    m = None
                elif level == 2:
                    b = (skey >> 10) & 0x7FF
                    m = (skey >> 21) == pvec
                else:
                    b = skey & 0x3FF
                    m = (skey >> 10) == pvec
                idx = b * C + lanes
                plsc.addupdate_scatter(hcnt, [idx], ones, mask=m)
                plsc.addupdate_scatter(hsum, [idx], v, mask=m)
                return carry

            lax.fori_loop(0, CH, rb, 0)

        bufs = (buf0, buf1)
        sems = (sem0, sem1)
        cps = [None, None]
        cps[0] = pltpu.async_copy(x_hbm.at[pl.ds(base, CH * C)], buf0, sem0)
        for g in range(NCH):
            cur = g & 1
            cps[cur].wait()
            if g + 1 < NCH:
                nxt = (g + 1) & 1
                cps[nxt] = pltpu.async_copy(
                    x_hbm.at[pl.ds(base + (g + 1) * CH * C, CH * C)],
                    bufs[nxt], sems[nxt])
            do_rows(bufs[cur])

        pltpu.sync_copy(hcnt, cnt_hbm.at[wid])
        pltpu.sync_copy(hsum, sum_hbm.at[wid])

    return functools.partial(
        pl.kernel, mesh=mesh, out_type=out_type, scratch_types=scratch,
        compiler_params=pltpu.CompilerParams(needs_layout_passes=False),
    )(body)


_pass1 = _make_pass(1, NB1)
_pass2 = _make_pass(2, NB2)
_pass3 = _make_pass(3, NB3)


def _cumsum0(a):
    s = 1
    n = a.shape[0]
    while s < n:
        a = a + jnp.concatenate(
            [jnp.zeros((s, a.shape[1]), a.dtype), a[:-s]], axis=0)
        s *= 2
    return a


def _scan_level(cnt, sm, kk):
    cum = _cumsum0(cnt)
    below = cum < kk[None, :]
    b = jnp.sum(below.astype(jnp.int32), axis=0)
    cntb = jnp.sum(jnp.where(below, cnt, 0.0), axis=0)
    sumb = jnp.sum(jnp.where(below, sm, 0.0), axis=0)
    return b, cntb, sumb


def _tail1(cnt, sm, extra, outs):
    kk = jnp.full((C,), float(K), jnp.float32)
    b, cntb, sumb = _scan_level(cnt, sm, kk)
    outs[0][...] = jnp.broadcast_to(b - 1024, (8, C))
    outs[1][...] = jnp.broadcast_to(kk - cntb, (8, C))
    outs[2][...] = jnp.broadcast_to(sumb, (8, C))


def _tail2(cnt, sm, extra, outs):
    pin, kkin, accin = extra
    kk = kkin[0]
    b, cntb, sumb = _scan_level(cnt, sm, kk)
    outs[0][...] = jnp.broadcast_to((pin[0] << 11) | b, (8, C))
    outs[1][...] = jnp.broadcast_to(kk - cntb, (8, C))
    outs[2][...] = jnp.broadcast_to(accin[0] + sumb, (8, C))


def _tail3(cnt, sm, extra, outs):
    pin, kkin, accin = extra
    kk = kkin[0]
    b, cntb, sumb = _scan_level(cnt, sm, kk)
    skt = (pin[0] << 10) | b
    ut = jnp.where(skt >= 0, skt, skt ^ 0x7FFFFFFF)
    t = lax.bitcast_convert_type(ut, jnp.float32)
    rem = kk - cntb
    total = accin[0] + sumb + rem * t
    outs[0][...] = jnp.broadcast_to(-(total / float(K)), (8, C))


_i32_8C = jax.ShapeDtypeStruct((8, C), jnp.int32)
_f32_8C = jax.ShapeDtypeStruct((8, C), jnp.float32)
_SMALL = pl.BlockSpec((8, C), lambda i: (0, 0))


def _make_merge(nb, n_extra, out_shapes, tailfn):
    n_out = len(out_shapes)

    def body(*refs):
        cnt_ref, sum_ref = refs[0], refs[1]
        extra = refs[2:2 + n_extra]
        outs = refs[2 + n_extra:2 + n_extra + n_out]
        acc_cnt, acc_sum = refs[-2], refs[-1]
        i = pl.program_id(0)

        @pl.when(i == 0)
        def _init():
            acc_cnt[...] = cnt_ref[0]
            acc_sum[...] = sum_ref[0]

        @pl.when(i > 0)
        def _accum():
            acc_cnt[...] += cnt_ref[0]
            acc_sum[...] += sum_ref[0]

        @pl.when(i == NW - 1)
        def _tail():
            tailfn(acc_cnt[...], acc_sum[...], extra, outs)

    hist_spec = pl.BlockSpec((1, nb, C), lambda i: (i, 0, 0))
    return pl.pallas_call(
        body,
        grid=(NW,),
        in_specs=[hist_spec, hist_spec] + [_SMALL] * n_extra,
        out_specs=[_SMALL] * n_out,
        out_shape=list(out_shapes),
        scratch_shapes=[pltpu.VMEM((nb, C), jnp.float32)] * 2,
    )


_merge1 = _make_merge(NB1, 0, [_i32_8C, _f32_8C, _f32_8C], _tail1)
_merge2 = _make_merge(NB2, 3, [_i32_8C, _f32_8C, _f32_8C], _tail2)
_merge3 = _make_merge(NB3, 3, [_f32_8C], _tail3)


def _hist3(a, nb):
    return a.reshape(NW, nb, C)


def kernel(input):
    xf = input.reshape(-1)
    p0 = jnp.zeros((C,), jnp.int32)
    c1, s1 = _pass1(xf, p0)
    p1, kk1, acc1 = _merge1(_hist3(c1, NB1), _hist3(s1, NB1))
    c2, s2 = _pass2(xf, p1[0])
    p2, kk2, acc2 = _merge2(_hist3(c2, NB2), _hist3(s2, NB2), p1, kk1, acc1)
    c3, s3 = _pass3(xf, p2[0])
    out8, = _merge3(_hist3(c3, NB3), _hist3(s3, NB3), p2, kk2, acc2)
    return out8[0]


# speedup vs baseline: 18.7858x; 1.8029x over previous
import functools

import jax
import jax.numpy as jnp
from jax import lax
from jax.experimental import pallas as pl
from jax.experimental.pallas import tpu as pltpu
from jax.experimental.pallas import tpu_sc as plsc

N = 1048576
C = 16
K = 104858
NC = 2
NS = 16
NW = NC * NS
ROWS_PER = N // NW
CH = 1024
NCH = ROWS_PER // CH

NB1 = 2048
NB2 = 2048
NB3 = 1024


def _make_pass(level, nb):
    mesh = plsc.VectorSubcoreMesh(core_axis_name="c", subcore_axis_name="s")
    out_type = [
        jax.ShapeDtypeStruct((NW, nb * C), jnp.float32),
        jax.ShapeDtypeStruct((NW, nb * C), jnp.float32),
    ]
    scratch = [
        pltpu.VMEM((CH * C,), jnp.float32),
        pltpu.VMEM((CH * C,), jnp.float32),
        pltpu.VMEM((nb * C,), jnp.float32),
        pltpu.VMEM((nb * C,), jnp.float32),
        pltpu.VMEM((C,), jnp.int32),
        pltpu.SemaphoreType.DMA,
        pltpu.SemaphoreType.DMA,
    ]

    def body(x_hbm, p_hbm, cnt_hbm, sum_hbm, buf0, buf1, hcnt, hsum, pv,
             sem0, sem1):
        cid = lax.axis_index("c")
        sid = lax.axis_index("s")
        wid = sid * NC + cid
        base = wid * ROWS_PER * C

        zero = jnp.zeros((C,), jnp.float32)
        ZU = 8

        def zb(i, carry):
            for j in range(ZU):
                hcnt[pl.ds(i * (ZU * C) + j * C, C)] = zero
                hsum[pl.ds(i * (ZU * C) + j * C, C)] = zero
            return carry

        lax.fori_loop(0, nb // ZU, zb, 0)

        pltpu.sync_copy(p_hbm, pv)
        pvec = pv[...]
        lanes = lax.iota(jnp.int32, C)
        ones = jnp.full((C,), 1.0, jnp.float32)

        RU = 8

        def do_rows(buf):
            @plsc.parallel_loop(0, CH, step=1, unroll=RU)
            def _rows(i):
                v = buf[pl.ds(i * C, C)]
                u = lax.bitcast_convert_type(v, jnp.int32)
                if level == 1:
                    b = (u >> 21) & 0x7FF
                    m = None
                elif level == 2:
                    b = (u >> 10) & 0x7FF
                    m = (u >> 21) == pvec
                else:
                    b = u & 0x3FF
                    m = (u >> 10) == pvec
                idx = b * C + lanes
                plsc.addupdate_scatter(hcnt, [idx], ones, mask=m)
                plsc.addupdate_scatter(hsum, [idx], v, mask=m)

        bufs = (buf0, buf1)
        sems = (sem0, sem1)
        cps = [None, None]
        cps[0] = pltpu.async_copy(x_hbm.at[pl.ds(base, CH * C)], buf0, sem0)
        for g in range(NCH):
            cur = g & 1
            cps[cur].wait()
            if g + 1 < NCH:
                nxt = (g + 1) & 1
                cps[nxt] = pltpu.async_copy(
                    x_hbm.at[pl.ds(base + (g + 1) * CH * C, CH * C)],
                    bufs[nxt], sems[nxt])
            do_rows(bufs[cur])

        pltpu.sync_copy(hcnt, cnt_hbm.at[wid])
        pltpu.sync_copy(hsum, sum_hbm.at[wid])

    return functools.partial(
        pl.kernel, mesh=mesh, out_type=out_type, scratch_types=scratch,
        compiler_params=pltpu.CompilerParams(needs_layout_passes=False),
    )(body)


_pass1 = _make_pass(1, NB1)
_pass2 = _make_pass(2, NB2)
_pass3 = _make_pass(3, NB3)


def _cumsum0(a):
    s = 1
    n = a.shape[0]
    while s < n:
        a = a + jnp.concatenate(
            [jnp.zeros((s, a.shape[1]), a.dtype), a[:-s]], axis=0)
        s *= 2
    return a


def _pick(G, cnt, sm, kk):
    below = G < kk[None, :]
    b = jnp.sum(below.astype(jnp.int32), axis=0)
    cntb = jnp.sum(jnp.where(below, cnt, 0.0), axis=0)
    sumb = jnp.sum(jnp.where(below, sm, 0.0), axis=0)
    return b, cntb, sumb


def _tail1(cnt, sm, extra, outs):
    H = NB1 // 2
    cum = _cumsum0(cnt)
    total = cum[NB1 - 1:NB1, :]
    sneg = total - cum[H - 1:H, :]
    j = lax.broadcasted_iota(jnp.int32, cnt.shape, 0)
    G = jnp.where(j >= H, total - cum + cnt, sneg + cum)
    kk = jnp.full((C,), float(K), jnp.float32)
    b_pos, cntb, sumb = _pick(G, cnt, sm, kk)
    praw = jnp.where(b_pos < H, (NB1 - 1) - b_pos, b_pos - H)
    pvec = jnp.where(praw >= H, praw - NB1, praw)
    outs[0][...] = jnp.broadcast_to(pvec, (8, C))
    outs[1][...] = jnp.broadcast_to(kk - cntb, (8, C))
    outs[2][...] = jnp.broadcast_to(sumb, (8, C))


def _dir_scan(cnt, sm, neg, nb, kk):
    cum = _cumsum0(cnt)
    total = cum[nb - 1:nb, :]
    G = jnp.where(neg[None, :], total - cum + cnt, cum)
    b_pos, cntb, sumb = _pick(G, cnt, sm, kk)
    b_raw = jnp.where(neg, (nb - 1) - b_pos, b_pos)
    return b_raw, cntb, sumb


def _tail2(cnt, sm, extra, outs):
    pin, kkin, accin = extra
    kk = kkin[0]
    neg = pin[0] < 0
    b_raw, cntb, sumb = _dir_scan(cnt, sm, neg, NB2, kk)
    outs[0][...] = jnp.broadcast_to((pin[0] << 11) | b_raw, (8, C))
    outs[1][...] = jnp.broadcast_to(kk - cntb, (8, C))
    outs[2][...] = jnp.broadcast_to(accin[0] + sumb, (8, C))


def _tail3(cnt, sm, extra, outs):
    pin, kkin, accin = extra
    kk = kkin[0]
    neg = pin[0] < 0
    b_raw, cntb, sumb = _dir_scan(cnt, sm, neg, NB3, kk)
    ut = (pin[0] << 10) | b_raw
    t = lax.bitcast_convert_type(ut, jnp.float32)
    rem = kk - cntb
    total = accin[0] + sumb + rem * t
    outs[0][...] = jnp.broadcast_to(-(total / float(K)), (8, C))


_i32_8C = jax.ShapeDtypeStruct((8, C), jnp.int32)
_f32_8C = jax.ShapeDtypeStruct((8, C), jnp.float32)
_SMALL = pl.BlockSpec((8, C), lambda i: (0, 0))


def _make_merge(nb, n_extra, out_shapes, tailfn):
    n_out = len(out_shapes)

    def body(*refs):
        cnt_ref, sum_ref = refs[0], refs[1]
        extra = refs[2:2 + n_extra]
        outs = refs[2 + n_extra:2 + n_extra + n_out]
        acc_cnt, acc_sum = refs[-2], refs[-1]
        i = pl.program_id(0)

        @pl.when(i == 0)
        def _init():
            acc_cnt[...] = cnt_ref[0]
            acc_sum[...] = sum_ref[0]

        @pl.when(i > 0)
        def _accum():
            acc_cnt[...] += cnt_ref[0]
            acc_sum[...] += sum_ref[0]

        @pl.when(i == NW - 1)
        def _tail():
            tailfn(acc_cnt[...], acc_sum[...], extra, outs)

    hist_spec = pl.BlockSpec((1, nb, C), lambda i: (i, 0, 0))
    return pl.pallas_call(
        body,
        grid=(NW,),
        in_specs=[hist_spec, hist_spec] + [_SMALL] * n_extra,
        out_specs=[_SMALL] * n_out,
        out_shape=list(out_shapes),
        scratch_shapes=[pltpu.VMEM((nb, C), jnp.float32)] * 2,
    )


_merge1 = _make_merge(NB1, 0, [_i32_8C, _f32_8C, _f32_8C], _tail1)
_merge2 = _make_merge(NB2, 3, [_i32_8C, _f32_8C, _f32_8C], _tail2)
_merge3 = _make_merge(NB3, 3, [_f32_8C], _tail3)


def _hist3(a, nb):
    return a.reshape(NW, nb, C)


def kernel(input):
    xf = input.reshape(-1)
    p0 = jnp.zeros((C,), jnp.int32)
    c1, s1 = _pass1(xf, p0)
    p1, kk1, acc1 = _merge1(_hist3(c1, NB1), _hist3(s1, NB1))
    c2, s2 = _pass2(xf, p1[0])
    p2, kk2, acc2 = _merge2(_hist3(c2, NB2), _hist3(s2, NB2), p1, kk1, acc1)
    c3, s3 = _pass3(xf, p2[0])
    out8, = _merge3(_hist3(c3, NB3), _hist3(s3, NB3), p2, kk2, acc2)
    return out8[0]


# speedup vs baseline: 22.3640x; 1.1905x over previous
import functools

import jax
import jax.numpy as jnp
from jax import lax
from jax.experimental import pallas as pl
from jax.experimental.pallas import tpu as pltpu
from jax.experimental.pallas import tpu_sc as plsc

N = 1048576
C = 16
K = 104858
NC = 2
NS = 16
NW = NC * NS
ROWS_PER = N // NW
CH = 1024
NCH = ROWS_PER // CH

NB1 = 2048
NB2 = 2048
NB3 = 1024
H1 = NB1 // 2


def _f32c(v):
    return jnp.full((C,), v, jnp.float32)


def _sc_merge(nbp, level, sid, bufs, sems, mslice, tmpw, tmpall, shst,
              histc_hbm, hists_hbm, pvec_in, kk_in, acc_in):
    SL = nbp * C // NS
    NBS = SL // C
    buf0, buf1 = bufs
    sem0, sem1 = sems

    srcs = ([histc_hbm.at[w, pl.ds(sid * SL, SL)] for w in range(NW)]
            + [hists_hbm.at[w, pl.ds(sid * SL, SL)] for w in range(NW)])
    bs = (buf0.at[pl.ds(0, SL)], buf1.at[pl.ds(0, SL)])
    nt = len(srcs)
    cps = [None, None]
    cps[0] = pltpu.async_copy(srcs[0], bs[0], sem0)
    for t in range(nt):
        cur = t & 1
        cps[cur].wait()
        if t + 1 < nt:
            nxt = (t + 1) & 1
            cps[nxt] = pltpu.async_copy(srcs[t + 1], bs[nxt],
                                        (sem0, sem1)[nxt])
        off = 0 if t < NW else SL
        first = t % NW == 0
        src_b = (buf0, buf1)[cur]

        if first:
            @plsc.parallel_loop(0, NBS, unroll=8)
            def _cp(i):
                mslice[pl.ds(off + i * C, C)] = src_b[pl.ds(i * C, C)]
        else:
            @plsc.parallel_loop(0, NBS, unroll=8)
            def _acc(i):
                mslice[pl.ds(off + i * C, C)] = (
                    mslice[pl.ds(off + i * C, C)] + src_b[pl.ds(i * C, C)])

    def tb(r, T):
        return T + mslice[pl.ds(r * C, C)]
    T = lax.fori_loop(0, NBS, tb, _f32c(0.0))

    tmpw[pl.ds(0, C)] = T
    pltpu.sync_copy(tmpw, shst.at[pl.ds(sid * 4 * C, 4 * C)])
    plsc.subcore_barrier()
    pltpu.sync_copy(shst, tmpall)

    sidf = sid.astype(jnp.float32)
    zc = _f32c(0.0)
    offA = zc
    offD = zc
    offPos = zc
    offNeg = zc
    sneg = zc
    total = zc
    for s in range(NS):
        Ts = tmpall[pl.ds(s * 4 * C, C)]
        lt = jnp.where(jnp.float32(s) < sidf, 1.0, 0.0)
        gt = jnp.where(jnp.float32(s) > sidf, 1.0, 0.0)
        total = total + Ts
        offA = offA + Ts * lt
        offD = offD + Ts * gt
        if level == 1:
            if s < NS // 2:
                offPos = offPos + Ts * lt
            else:
                sneg = sneg + Ts
                offNeg = offNeg + Ts * gt

    if level == 1:
        kk = _f32c(float(K))
        cneg = jnp.where(sidf >= jnp.float32(NS // 2), 1.0, 0.0)
        sel = _f32c(1.0) * cneg
        off = cneg * offNeg + (1.0 - cneg) * (sneg + offPos)
    else:
        kk = kk_in
        sel = jnp.where(pvec_in < 0, 1.0, 0.0)
        off = sel * offD + (1.0 - sel) * offA

    def sb(r, carry):
        cum, bp, cb, sb_ = carry
        c = mslice[pl.ds(r * C, C)]
        sm = mslice[pl.ds(SL + r * C, C)]
        cum = cum + c
        G = off + sel * (T - cum + c) + (1.0 - sel) * cum
        m = G < kk
        bp = bp + jnp.where(m, 1.0, 0.0)
        cb = cb + jnp.where(m, c, 0.0)
        sb_ = sb_ + jnp.where(m, sm, 0.0)
        return cum, bp, cb, sb_

    _, bp, cb, smb = lax.fori_loop(0, NBS, sb, (zc, zc, zc, zc))

    tmpw[pl.ds(C, C)] = bp
    tmpw[pl.ds(2 * C, C)] = cb
    tmpw[pl.ds(3 * C, C)] = smb
    pltpu.sync_copy(tmpw, shst.at[pl.ds(sid * 4 * C, 4 * C)])
    plsc.subcore_barrier()
    pltpu.sync_copy(shst, tmpall)

    bpg, cbg, smg = zc, zc, zc
    for s in range(NS):
        bpg = bpg + tmpall[pl.ds(s * 4 * C + C, C)]
        cbg = cbg + tmpall[pl.ds(s * 4 * C + 2 * C, C)]
        smg = smg + tmpall[pl.ds(s * 4 * C + 3 * C, C)]

    b_pos = bpg.astype(jnp.int32)
    if level == 1:
        praw = jnp.where(b_pos < H1, (NB1 - 1) - b_pos, b_pos - H1)
        pvec = jnp.where(praw >= H1, praw - NB1, praw)
    else:
        neg = pvec_in < 0
        b_raw = jnp.where(neg, (nbp - 1) - b_pos, b_pos)
        sh = 11 if level == 2 else 10
        pvec = (pvec_in << sh) | b_raw
    kk_out = kk - cbg
    acc = (zc if level == 1 else acc_in) + smg
    return pvec, kk_out, acc


def _make_pass(level, nb):
    mesh = plsc.VectorSubcoreMesh(core_axis_name="c", subcore_axis_name="s")
    nbp = {2: NB1, 3: NB2}.get(level)
    out_type = [
        jax.ShapeDtypeStruct((NW, nb * C), jnp.float32),
        jax.ShapeDtypeStruct((NW, nb * C), jnp.float32),
    ]
    if level > 1:
        out_type += [
            jax.ShapeDtypeStruct((C,), jnp.int32),
            jax.ShapeDtypeStruct((C,), jnp.float32),
            jax.ShapeDtypeStruct((C,), jnp.float32),
        ]
    scratch = [
        pltpu.VMEM((CH * C,), jnp.float32),
        pltpu.VMEM((CH * C,), jnp.float32),
        pltpu.VMEM((nb * C,), jnp.float32),
        pltpu.VMEM((nb * C,), jnp.float32),
        pltpu.SemaphoreType.DMA,
        pltpu.SemaphoreType.DMA,
    ]
    if level > 1:
        scratch += [
            pltpu.VMEM((2 * nbp * C // NS,), jnp.float32),
            pltpu.VMEM((4 * C,), jnp.float32),
            pltpu.VMEM((NS * 4 * C,), jnp.float32),
            pltpu.VMEM_SHARED((NS * 4 * C,), jnp.float32),
            pltpu.VMEM((C,), jnp.int32),
            pltpu.VMEM((C,), jnp.float32),
            pltpu.VMEM((C,), jnp.float32),
        ]

    def body(*refs):
        if level == 1:
            (x_hbm, cnt_hbm, sum_hbm,
             buf0, buf1, hcnt, hsum, sem0, sem1) = refs
        else:
            (x_hbm, pc_hbm, ps_hbm, pin_hbm, kin_hbm, ain_hbm,
             cnt_hbm, sum_hbm, p_out, k_out, a_out,
             buf0, buf1, hcnt, hsum, sem0, sem1,
             mslice, tmpw, tmpall, shst, pvm, kvm, avm) = refs
        cid = lax.axis_index("c")
        sid = lax.axis_index("s")
        wid = sid * NC + cid
        base = wid * ROWS_PER * C

        if level == 1:
            pvec = None
        else:
            if level == 2:
                pvi, kki, aci = None, None, None
            else:
                pltpu.sync_copy(pin_hbm, pvm)
                pltpu.sync_copy(kin_hbm, kvm)
                pltpu.sync_copy(ain_hbm, avm)
                pvi, kki, aci = pvm[...], kvm[...], avm[...]
            pvec, kk, acc = _sc_merge(
                nbp, level - 1, sid, (buf0, buf1), (sem0, sem1),
                mslice, tmpw, tmpall, shst, pc_hbm, ps_hbm, pvi, kki, aci)
            pvm[...] = pvec
            kvm[...] = kk
            avm[...] = acc

            @pl.when(wid == 0)
            def _stats():
                pltpu.sync_copy(pvm, p_out)
                pltpu.sync_copy(kvm, k_out)
                pltpu.sync_copy(avm, a_out)

        zero = jnp.zeros((C,), jnp.float32)
        ZU = 8

        def zb(i, carry):
            for j in range(ZU):
                hcnt[pl.ds(i * (ZU * C) + j * C, C)] = zero
                hsum[pl.ds(i * (ZU * C) + j * C, C)] = zero
            return carry

        lax.fori_loop(0, nb // ZU, zb, 0)

        lanes = lax.iota(jnp.int32, C)
        ones = jnp.full((C,), 1.0, jnp.float32)
        RU = 8

        def do_rows(buf):
            @plsc.parallel_loop(0, CH, step=1, unroll=RU)
            def _rows(i):
                v = buf[pl.ds(i * C, C)]
                u = lax.bitcast_convert_type(v, jnp.int32)
                if level == 1:
                    b = (u >> 21) & 0x7FF
                    m = None
                elif level == 2:
                    b = (u >> 10) & 0x7FF
                    m = (u >> 21) == pvec
                else:
                    b = u & 0x3FF
                    m = (u >> 10) == pvec
                idx = b * C + lanes
                plsc.addupdate_scatter(hcnt, [idx], ones, mask=m)
                plsc.addupdate_scatter(hsum, [idx], v, mask=m)

        bufs = (buf0, buf1)
        sems = (sem0, sem1)
        cps = [None, None]
        cps[0] = pltpu.async_copy(x_hbm.at[pl.ds(base, CH * C)], buf0, sem0)
        for g in range(NCH):
            cur = g & 1
            cps[cur].wait()
            if g + 1 < NCH:
                nxt = (g + 1) & 1
                cps[nxt] = pltpu.async_copy(
                    x_hbm.at[pl.ds(base + (g + 1) * CH * C, CH * C)],
                    bufs[nxt], sems[nxt])
            do_rows(bufs[cur])

        pltpu.sync_copy(hcnt, cnt_hbm.at[wid])
        pltpu.sync_copy(hsum, sum_hbm.at[wid])

    return functools.partial(
        pl.kernel, mesh=mesh, out_type=out_type, scratch_types=scratch,
        compiler_params=pltpu.CompilerParams(needs_layout_passes=False),
    )(body)


_pass1 = _make_pass(1, NB1)
_pass2 = _make_pass(2, NB2)
_pass3 = _make_pass(3, NB3)


def _make_final():
    mesh = plsc.VectorSubcoreMesh(core_axis_name="c", subcore_axis_name="s")
    SLW = 2 * NB3 * C // NS
    scratch = [
        pltpu.VMEM((NB3 * C // NS,), jnp.float32),
        pltpu.VMEM((NB3 * C // NS,), jnp.float32),
        pltpu.VMEM((SLW,), jnp.float32),
        pltpu.VMEM((4 * C,), jnp.float32),
        pltpu.VMEM((NS * 4 * C,), jnp.float32),
        pltpu.VMEM_SHARED((NS * 4 * C,), jnp.float32),
        pltpu.VMEM((C,), jnp.int32),
        pltpu.VMEM((C,), jnp.float32),
        pltpu.VMEM((C,), jnp.float32),
        pltpu.SemaphoreType.DMA,
        pltpu.SemaphoreType.DMA,
    ]

    def body(pc_hbm, ps_hbm, pin_hbm, kin_hbm, ain_hbm, out_hbm,
             buf0, buf1, mslice, tmpw, tmpall, shst, pvm, kvm, avm,
             sem0, sem1):
        cid = lax.axis_index("c")
        sid = lax.axis_index("s")
        wid = sid * NC + cid
        pltpu.sync_copy(pin_hbm, pvm)
        pltpu.sync_copy(kin_hbm, kvm)
        pltpu.sync_copy(ain_hbm, avm)
        ut, rem, acc = _sc_merge(
            NB3, 3, sid, (buf0, buf1), (sem0, sem1),
            mslice, tmpw, tmpall, shst, pc_hbm, ps_hbm,
            pvm[...], kvm[...], avm[...])
        t = lax.bitcast_convert_type(ut, jnp.float32)
        res = -((acc + rem * t) / float(K))
        kvm[...] = res

        @pl.when(wid == 0)
        def _w():
            pltpu.sync_copy(kvm, out_hbm)

    return functools.partial(
        pl.kernel, mesh=mesh,
        out_type=jax.ShapeDtypeStruct((C,), jnp.float32),
        scratch_types=scratch,
        compiler_params=pltpu.CompilerParams(needs_layout_passes=False),
    )(body)


_final = _make_final()


def kernel(input):
    xf = input.reshape(-1)
    zi = jnp.zeros((C,), jnp.int32)
    zf = jnp.zeros((C,), jnp.float32)
    c1, s1 = _pass1(xf)
    c2, s2, p1, kk1, a1 = _pass2(xf, c1, s1, zi, zf, zf)
    c3, s3, p2, kk2, a2 = _pass3(xf, c2, s2, p1, kk1, a1)
    return _final(c3, s3, p2, kk2, a2)


# speedup vs baseline: 25.1269x; 1.1235x over previous
import functools

import jax
import jax.numpy as jnp
from jax import lax
from jax.experimental import pallas as pl
from jax.experimental.pallas import tpu as pltpu
from jax.experimental.pallas import tpu_sc as plsc

N = 1048576
C = 16
K = 104858
NC = 2
NS = 16
NW = NC * NS
ROWS_PER = N // NW
CH = 1024
NCH = ROWS_PER // CH

NB1 = 2048
NB2 = 2048
NB3 = 1024
H1 = NB1 // 2


def _f32c(v):
    return jnp.full((C,), v, jnp.float32)


def _sc_merge(nbp, level, sid, ringbuf, sems, mslice, tmpw, tmpall, shst,
              histc_hbm, hists_hbm, pvec_in, kk_in, acc_in):
    SL = nbp * C // NS
    NBS = SL // C
    DEPTH = len(sems)

    srcs = ([histc_hbm.at[w, pl.ds(sid * SL, SL)] for w in range(NW)]
            + [hists_hbm.at[w, pl.ds(sid * SL, SL)] for w in range(NW)])
    slots = [ringbuf.at[pl.ds(d * SL, SL)] for d in range(DEPTH)]
    nt = len(srcs)
    cps = [None] * DEPTH
    for d in range(min(DEPTH, nt)):
        cps[d] = pltpu.async_copy(srcs[d], slots[d], sems[d])
    for t in range(nt):
        cur = t % DEPTH
        cps[cur].wait()
        off = 0 if t < NW else SL
        first = t % NW == 0
        soff = cur * SL

        if first:
            @plsc.parallel_loop(0, NBS, unroll=8)
            def _cp(i):
                mslice[pl.ds(off + i * C, C)] = ringbuf[pl.ds(soff + i * C, C)]
        else:
            @plsc.parallel_loop(0, NBS, unroll=8)
            def _acc(i):
                mslice[pl.ds(off + i * C, C)] = (
                    mslice[pl.ds(off + i * C, C)]
                    + ringbuf[pl.ds(soff + i * C, C)])

        if t + DEPTH < nt:
            cps[cur] = pltpu.async_copy(srcs[t + DEPTH], slots[cur],
                                        sems[cur])

    def tb(r, T):
        return T + mslice[pl.ds(r * C, C)]
    T = lax.fori_loop(0, NBS, tb, _f32c(0.0))

    tmpw[pl.ds(0, C)] = T
    pltpu.sync_copy(tmpw, shst.at[pl.ds(sid * 4 * C, 4 * C)])
    plsc.subcore_barrier()
    pltpu.sync_copy(shst, tmpall)

    sidf = sid.astype(jnp.float32)
    zc = _f32c(0.0)
    offA = zc
    offD = zc
    offPos = zc
    offNeg = zc
    sneg = zc
    total = zc
    for s in range(NS):
        Ts = tmpall[pl.ds(s * 4 * C, C)]
        lt = jnp.where(jnp.float32(s) < sidf, 1.0, 0.0)
        gt = jnp.where(jnp.float32(s) > sidf, 1.0, 0.0)
        total = total + Ts
        offA = offA + Ts * lt
        offD = offD + Ts * gt
        if level == 1:
            if s < NS // 2:
                offPos = offPos + Ts * lt
            else:
                sneg = sneg + Ts
                offNeg = offNeg + Ts * gt

    if level == 1:
        kk = _f32c(float(K))
        cneg = jnp.where(sidf >= jnp.float32(NS // 2), 1.0, 0.0)
        sel = _f32c(1.0) * cneg
        off = cneg * offNeg + (1.0 - cneg) * (sneg + offPos)
    else:
        kk = kk_in
        sel = jnp.where(pvec_in < 0, 1.0, 0.0)
        off = sel * offD + (1.0 - sel) * offA

    def sb(r, carry):
        cum, bp, cb, sb_ = carry
        c = mslice[pl.ds(r * C, C)]
        sm = mslice[pl.ds(SL + r * C, C)]
        cum = cum + c
        G = off + sel * (T - cum + c) + (1.0 - sel) * cum
        m = G < kk
        bp = bp + jnp.where(m, 1.0, 0.0)
        cb = cb + jnp.where(m, c, 0.0)
        sb_ = sb_ + jnp.where(m, sm, 0.0)
        return cum, bp, cb, sb_

    _, bp, cb, smb = lax.fori_loop(0, NBS, sb, (zc, zc, zc, zc))

    tmpw[pl.ds(C, C)] = bp
    tmpw[pl.ds(2 * C, C)] = cb
    tmpw[pl.ds(3 * C, C)] = smb
    pltpu.sync_copy(tmpw, shst.at[pl.ds(sid * 4 * C, 4 * C)])
    plsc.subcore_barrier()
    pltpu.sync_copy(shst, tmpall)

    bpg, cbg, smg = zc, zc, zc
    for s in range(NS):
        bpg = bpg + tmpall[pl.ds(s * 4 * C + C, C)]
        cbg = cbg + tmpall[pl.ds(s * 4 * C + 2 * C, C)]
        smg = smg + tmpall[pl.ds(s * 4 * C + 3 * C, C)]

    b_pos = bpg.astype(jnp.int32)
    if level == 1:
        praw = jnp.where(b_pos < H1, (NB1 - 1) - b_pos, b_pos - H1)
        pvec = jnp.where(praw >= H1, praw - NB1, praw)
    else:
        neg = pvec_in < 0
        b_raw = jnp.where(neg, (nbp - 1) - b_pos, b_pos)
        sh = 11 if level == 2 else 10
        pvec = (pvec_in << sh) | b_raw
    kk_out = kk - cbg
    acc = (zc if level == 1 else acc_in) + smg
    return pvec, kk_out, acc


def _make_pass(level, nb):
    mesh = plsc.VectorSubcoreMesh(core_axis_name="c", subcore_axis_name="s")
    nbp = {2: NB1, 3: NB2}.get(level)
    out_type = [
        jax.ShapeDtypeStruct((NW, nb * C), jnp.float32),
        jax.ShapeDtypeStruct((NW, nb * C), jnp.float32),
    ]
    if level > 1:
        out_type += [
            jax.ShapeDtypeStruct((C,), jnp.int32),
            jax.ShapeDtypeStruct((C,), jnp.float32),
            jax.ShapeDtypeStruct((C,), jnp.float32),
        ]
    scratch = [
        pltpu.VMEM((CH * C,), jnp.float32),
        pltpu.VMEM((CH * C,), jnp.float32),
        pltpu.VMEM((nb * C,), jnp.float32),
        pltpu.VMEM((nb * C,), jnp.float32),
        pltpu.SemaphoreType.DMA,
        pltpu.SemaphoreType.DMA,
    ]
    if level > 1:
        scratch += [
            pltpu.VMEM((2 * nbp * C // NS,), jnp.float32),
            pltpu.VMEM((4 * C,), jnp.float32),
            pltpu.VMEM((NS * 4 * C,), jnp.float32),
            pltpu.VMEM_SHARED((NS * 4 * C,), jnp.float32),
            pltpu.VMEM((C,), jnp.int32),
            pltpu.VMEM((C,), jnp.float32),
            pltpu.VMEM((C,), jnp.float32),
            pltpu.SemaphoreType.DMA,
            pltpu.SemaphoreType.DMA,
        ]

    def body(*refs):
        if level == 1:
            (x_hbm, cnt_hbm, sum_hbm,
             buf0, buf1, hcnt, hsum, sem0, sem1) = refs
        else:
            (x_hbm, pc_hbm, ps_hbm, pin_hbm, kin_hbm, ain_hbm,
             cnt_hbm, sum_hbm, p_out, k_out, a_out,
             buf0, buf1, hcnt, hsum, sem0, sem1,
             mslice, tmpw, tmpall, shst, pvm, kvm, avm,
             sem2, sem3) = refs
        cid = lax.axis_index("c")
        sid = lax.axis_index("s")
        wid = sid * NC + cid
        base = wid * ROWS_PER * C

        if level == 1:
            pvec = None
        else:
            if level == 2:
                pvi, kki, aci = None, None, None
            else:
                pltpu.sync_copy(pin_hbm, pvm)
                pltpu.sync_copy(kin_hbm, kvm)
                pltpu.sync_copy(ain_hbm, avm)
                pvi, kki, aci = pvm[...], kvm[...], avm[...]
            pvec, kk, acc = _sc_merge(
                nbp, level - 1, sid, buf0, (sem0, sem1, sem2, sem3),
                mslice, tmpw, tmpall, shst, pc_hbm, ps_hbm, pvi, kki, aci)
            pvm[...] = pvec
            kvm[...] = kk
            avm[...] = acc

            @pl.when(wid == 0)
            def _stats():
                pltpu.sync_copy(pvm, p_out)
                pltpu.sync_copy(kvm, k_out)
                pltpu.sync_copy(avm, a_out)

        zero = jnp.zeros((C,), jnp.float32)
        ZU = 8

        def zb(i, carry):
            for j in range(ZU):
                hcnt[pl.ds(i * (ZU * C) + j * C, C)] = zero
                hsum[pl.ds(i * (ZU * C) + j * C, C)] = zero
            return carry

        lax.fori_loop(0, nb // ZU, zb, 0)

        lanes = lax.iota(jnp.int32, C)
        ones = jnp.full((C,), 1.0, jnp.float32)
        RU = 8

        def do_rows(buf):
            @plsc.parallel_loop(0, CH, step=1, unroll=RU)
            def _rows(i):
                v = buf[pl.ds(i * C, C)]
                u = lax.bitcast_convert_type(v, jnp.int32)
                if level == 1:
                    b = (u >> 21) & 0x7FF
                    m = None
                elif level == 2:
                    b = (u >> 10) & 0x7FF
                    m = (u >> 21) == pvec
                else:
                    b = u & 0x3FF
                    m = (u >> 10) == pvec
                idx = b * C + lanes
                plsc.addupdate_scatter(hcnt, [idx], ones, mask=m)
                plsc.addupdate_scatter(hsum, [idx], v, mask=m)

        bufs = (buf0, buf1)
        sems = (sem0, sem1)
        cps = [None, None]
        cps[0] = pltpu.async_copy(x_hbm.at[pl.ds(base, CH * C)], buf0, sem0)
        for g in range(NCH):
            cur = g & 1
            cps[cur].wait()
            if g + 1 < NCH:
                nxt = (g + 1) & 1
                cps[nxt] = pltpu.async_copy(
                    x_hbm.at[pl.ds(base + (g + 1) * CH * C, CH * C)],
                    bufs[nxt], sems[nxt])
            do_rows(bufs[cur])

        pltpu.sync_copy(hcnt, cnt_hbm.at[wid])
        pltpu.sync_copy(hsum, sum_hbm.at[wid])

    return functools.partial(
        pl.kernel, mesh=mesh, out_type=out_type, scratch_types=scratch,
        compiler_params=pltpu.CompilerParams(needs_layout_passes=False),
    )(body)


_pass1 = _make_pass(1, NB1)
_pass2 = _make_pass(2, NB2)
_pass3 = _make_pass(3, NB3)


def _make_final():
    mesh = plsc.VectorSubcoreMesh(core_axis_name="c", subcore_axis_name="s")
    SLW = 2 * NB3 * C // NS
    scratch = [
        pltpu.VMEM((4 * NB3 * C // NS,), jnp.float32),
        pltpu.VMEM((SLW,), jnp.float32),
        pltpu.VMEM((4 * C,), jnp.float32),
        pltpu.VMEM((NS * 4 * C,), jnp.float32),
        pltpu.VMEM_SHARED((NS * 4 * C,), jnp.float32),
        pltpu.VMEM((C,), jnp.int32),
        pltpu.VMEM((C,), jnp.float32),
        pltpu.VMEM((C,), jnp.float32),
        pltpu.SemaphoreType.DMA,
        pltpu.SemaphoreType.DMA,
        pltpu.SemaphoreType.DMA,
        pltpu.SemaphoreType.DMA,
    ]

    def body(pc_hbm, ps_hbm, pin_hbm, kin_hbm, ain_hbm, out_hbm,
             ring, mslice, tmpw, tmpall, shst, pvm, kvm, avm,
             sem0, sem1, sem2, sem3):
        cid = lax.axis_index("c")
        sid = lax.axis_index("s")
        wid = sid * NC + cid
        pltpu.sync_copy(pin_hbm, pvm)
        pltpu.sync_copy(kin_hbm, kvm)
        pltpu.sync_copy(ain_hbm, avm)
        ut, rem, acc = _sc_merge(
            NB3, 3, sid, ring, (sem0, sem1, sem2, sem3),
            mslice, tmpw, tmpall, shst, pc_hbm, ps_hbm,
            pvm[...], kvm[...], avm[...])
        t = lax.bitcast_convert_type(ut, jnp.float32)
        res = -((acc + rem * t) / float(K))
        kvm[...] = res

        @pl.when(wid == 0)
        def _w():
            pltpu.sync_copy(kvm, out_hbm)

    return functools.partial(
        pl.kernel, mesh=mesh,
        out_type=jax.ShapeDtypeStruct((C,), jnp.float32),
        scratch_types=scratch,
        compiler_params=pltpu.CompilerParams(needs_layout_passes=False),
    )(body)


_final = _make_final()


def kernel(input):
    xf = input.reshape(-1)
    zi = jnp.zeros((C,), jnp.int32)
    zf = jnp.zeros((C,), jnp.float32)
    c1, s1 = _pass1(xf)
    c2, s2, p1, kk1, a1 = _pass2(xf, c1, s1, zi, zf, zf)
    c3, s3, p2, kk2, a2 = _pass3(xf, c2, s2, p1, kk1, a1)
    return _final(c3, s3, p2, kk2, a2)


# speedup vs baseline: 25.2859x; 1.0063x over previous
import functools

import jax
import jax.numpy as jnp
from jax import lax
from jax.experimental import pallas as pl
from jax.experimental.pallas import tpu as pltpu
from jax.experimental.pallas import tpu_sc as plsc

N = 1048576
C = 16
K = 104858
NC = 2
NS = 16
NW = NC * NS
ROWS_PER = N // NW
CH = 1024
NCH = ROWS_PER // CH

NB1 = 2048
NB2 = 2048
NB3 = 1024
H1 = NB1 // 2


def _f32c(v):
    return jnp.full((C,), v, jnp.float32)


def _sc_merge(nbp, level, sid, ringbuf, sems, mslice, tmpw, tmpall, shst,
              histc_hbm, hists_hbm, pvec_in, kk_in, acc_in):
    SL = nbp * C // NS
    NBS = SL // C
    DEPTH = len(sems)

    srcs = ([histc_hbm.at[w, pl.ds(sid * SL, SL)] for w in range(NW)]
            + [hists_hbm.at[w, pl.ds(sid * SL, SL)] for w in range(NW)])
    slots = [ringbuf.at[pl.ds(d * SL, SL)] for d in range(DEPTH)]
    nt = len(srcs)
    cps = [None] * DEPTH
    for d in range(min(DEPTH, nt)):
        cps[d] = pltpu.async_copy(srcs[d], slots[d], sems[d])
    for t in range(nt):
        cur = t % DEPTH
        cps[cur].wait()
        off = 0 if t < NW else SL
        first = t % NW == 0
        soff = cur * SL

        if first:
            @plsc.parallel_loop(0, NBS, unroll=8)
            def _cp(i):
                mslice[pl.ds(off + i * C, C)] = ringbuf[pl.ds(soff + i * C, C)]
        else:
            @plsc.parallel_loop(0, NBS, unroll=8)
            def _acc(i):
                mslice[pl.ds(off + i * C, C)] = (
                    mslice[pl.ds(off + i * C, C)]
                    + ringbuf[pl.ds(soff + i * C, C)])

        if t + DEPTH < nt:
            cps[cur] = pltpu.async_copy(srcs[t + DEPTH], slots[cur],
                                        sems[cur])

    def tb(r, T):
        return T + mslice[pl.ds(r * C, C)]
    T = lax.fori_loop(0, NBS, tb, _f32c(0.0))

    tmpw[pl.ds(0, C)] = T
    pltpu.sync_copy(tmpw, shst.at[pl.ds(sid * 4 * C, 4 * C)])
    plsc.subcore_barrier()
    pltpu.sync_copy(shst, tmpall)

    sidf = sid.astype(jnp.float32)
    zc = _f32c(0.0)
    offA = zc
    offD = zc
    offPos = zc
    offNeg = zc
    sneg = zc
    total = zc
    for s in range(NS):
        Ts = tmpall[pl.ds(s * 4 * C, C)]
        lt = jnp.where(jnp.float32(s) < sidf, 1.0, 0.0)
        gt = jnp.where(jnp.float32(s) > sidf, 1.0, 0.0)
        total = total + Ts
        offA = offA + Ts * lt
        offD = offD + Ts * gt
        if level == 1:
            if s < NS // 2:
                offPos = offPos + Ts * lt
            else:
                sneg = sneg + Ts
                offNeg = offNeg + Ts * gt

    if level == 1:
        kk = _f32c(float(K))
        cneg = jnp.where(sidf >= jnp.float32(NS // 2), 1.0, 0.0)
        sel = _f32c(1.0) * cneg
        off = cneg * offNeg + (1.0 - cneg) * (sneg + offPos)
    else:
        kk = kk_in
        sel = jnp.where(pvec_in < 0, 1.0, 0.0)
        off = sel * offD + (1.0 - sel) * offA

    def sb(r, carry):
        cum, bp, cb, sb_ = carry
        c = mslice[pl.ds(r * C, C)]
        sm = mslice[pl.ds(SL + r * C, C)]
        cum = cum + c
        G = off + sel * (T - cum + c) + (1.0 - sel) * cum
        m = G < kk
        bp = bp + jnp.where(m, 1.0, 0.0)
        cb = cb + jnp.where(m, c, 0.0)
        sb_ = sb_ + jnp.where(m, sm, 0.0)
        return cum, bp, cb, sb_

    _, bp, cb, smb = lax.fori_loop(0, NBS, sb, (zc, zc, zc, zc))

    tmpw[pl.ds(C, C)] = bp
    tmpw[pl.ds(2 * C, C)] = cb
    tmpw[pl.ds(3 * C, C)] = smb
    pltpu.sync_copy(tmpw, shst.at[pl.ds(sid * 4 * C, 4 * C)])
    plsc.subcore_barrier()
    pltpu.sync_copy(shst, tmpall)

    bpg, cbg, smg = zc, zc, zc
    for s in range(NS):
        bpg = bpg + tmpall[pl.ds(s * 4 * C + C, C)]
        cbg = cbg + tmpall[pl.ds(s * 4 * C + 2 * C, C)]
        smg = smg + tmpall[pl.ds(s * 4 * C + 3 * C, C)]

    b_pos = bpg.astype(jnp.int32)
    if level == 1:
        praw = jnp.where(b_pos < H1, (NB1 - 1) - b_pos, b_pos - H1)
        pvec = jnp.where(praw >= H1, praw - NB1, praw)
    else:
        neg = pvec_in < 0
        b_raw = jnp.where(neg, (nbp - 1) - b_pos, b_pos)
        sh = 11 if level == 2 else 10
        pvec = (pvec_in << sh) | b_raw
    kk_out = kk - cbg
    acc = (zc if level == 1 else acc_in) + smg
    return pvec, kk_out, acc


def _make_pass(level, nb):
    mesh = plsc.VectorSubcoreMesh(core_axis_name="c", subcore_axis_name="s")
    nbp = {2: NB1, 3: NB2}.get(level)
    out_type = [
        jax.ShapeDtypeStruct((NW, nb * C), jnp.float32),
        jax.ShapeDtypeStruct((NW, nb * C), jnp.float32),
    ]
    if level > 1:
        out_type += [
            jax.ShapeDtypeStruct((C,), jnp.int32),
            jax.ShapeDtypeStruct((C,), jnp.float32),
            jax.ShapeDtypeStruct((C,), jnp.float32),
        ]
    scratch = [
        pltpu.VMEM((CH * C,), jnp.float32),
        pltpu.VMEM((CH * C,), jnp.float32),
        pltpu.VMEM((nb * C,), jnp.float32),
        pltpu.VMEM((nb * C,), jnp.float32),
        pltpu.SemaphoreType.DMA,
        pltpu.SemaphoreType.DMA,
    ]
    if level > 1:
        scratch += [
            pltpu.VMEM((2 * nbp * C // NS,), jnp.float32),
            pltpu.VMEM((4 * C,), jnp.float32),
            pltpu.VMEM((NS * 4 * C,), jnp.float32),
            pltpu.VMEM_SHARED((NS * 4 * C,), jnp.float32),
            pltpu.VMEM((C,), jnp.int32),
            pltpu.VMEM((C,), jnp.float32),
            pltpu.VMEM((C,), jnp.float32),
            pltpu.SemaphoreType.DMA,
            pltpu.SemaphoreType.DMA,
        ]

    def body(*refs):
        if level == 1:
            (x_hbm, cnt_hbm, sum_hbm,
             buf0, buf1, hcnt, hsum, sem0, sem1) = refs
        else:
            (x_hbm, pc_hbm, ps_hbm, pin_hbm, kin_hbm, ain_hbm,
             cnt_hbm, sum_hbm, p_out, k_out, a_out,
             buf0, buf1, hcnt, hsum, sem0, sem1,
             mslice, tmpw, tmpall, shst, pvm, kvm, avm,
             sem2, sem3) = refs
        cid = lax.axis_index("c")
        sid = lax.axis_index("s")
        wid = sid * NC + cid
        base = wid * ROWS_PER * C

        if level == 1:
            pvec = None
        else:
            if level == 2:
                pvi, kki, aci = None, None, None
            else:
                pltpu.sync_copy(pin_hbm, pvm)
                pltpu.sync_copy(kin_hbm, kvm)
                pltpu.sync_copy(ain_hbm, avm)
                pvi, kki, aci = pvm[...], kvm[...], avm[...]
            pvec, kk, acc = _sc_merge(
                nbp, level - 1, sid, buf0, (sem0, sem1, sem2, sem3),
                mslice, tmpw, tmpall, shst, pc_hbm, ps_hbm, pvi, kki, aci)
            pvm[...] = pvec
            kvm[...] = kk
            avm[...] = acc

            @pl.when(wid == 0)
            def _stats():
                pltpu.sync_copy(pvm, p_out)
                pltpu.sync_copy(kvm, k_out)
                pltpu.sync_copy(avm, a_out)

        zero = jnp.zeros((C,), jnp.float32)
        ZU = 8

        def zb(i, carry):
            for j in range(ZU):
                hcnt[pl.ds(i * (ZU * C) + j * C, C)] = zero
                hsum[pl.ds(i * (ZU * C) + j * C, C)] = zero
            return carry

        lax.fori_loop(0, nb // ZU, zb, 0)

        lanes = lax.iota(jnp.int32, C)
        ones = jnp.full((C,), 1.0, jnp.float32)
        RU = 8

        def do_rows(buf):
            @plsc.parallel_loop(0, CH, step=1, unroll=RU)
            def _rows(i):
                v = buf[pl.ds(i * C, C)]
                u = lax.bitcast_convert_type(v, jnp.int32)
                if level == 1:
                    b = (u >> 21) & 0x7FF
                    m = None
                elif level == 2:
                    b = (u >> 10) & 0x7FF
                    m = (u >> 21) == pvec
                else:
                    b = u & 0x3FF
                    m = (u >> 10) == pvec
                idx = b * C + lanes
                plsc.addupdate_scatter(hcnt, [idx], ones, mask=m)
                plsc.addupdate_scatter(hsum, [idx], v, mask=m)

        CHW = CH * C

        def src(g):
            return x_hbm.at[pl.ds(base + g * CHW, CHW)]

        pltpu.async_copy(src(0), buf0, sem0)
        pltpu.async_copy(src(1), buf1, sem1)

        def pair(g2, carry):
            g = 2 * g2
            pltpu.make_async_copy(src(g), buf0, sem0).wait()
            do_rows(buf0)
            pltpu.async_copy(src(g + 2), buf0, sem0)
            pltpu.make_async_copy(src(g + 1), buf1, sem1).wait()
            do_rows(buf1)
            pltpu.async_copy(src(g + 3), buf1, sem1)
            return carry

        lax.fori_loop(0, NCH // 2 - 2, pair, 0)
        g = NCH - 4
        pltpu.make_async_copy(src(g), buf0, sem0).wait()
        do_rows(buf0)
        pltpu.async_copy(src(g + 2), buf0, sem0)
        pltpu.make_async_copy(src(g + 1), buf1, sem1).wait()
        do_rows(buf1)
        pltpu.async_copy(src(g + 3), buf1, sem1)
        pltpu.make_async_copy(src(g + 2), buf0, sem0).wait()
        do_rows(buf0)
        pltpu.make_async_copy(src(g + 3), buf1, sem1).wait()
        do_rows(buf1)

        pltpu.sync_copy(hcnt, cnt_hbm.at[wid])
        pltpu.sync_copy(hsum, sum_hbm.at[wid])

    return functools.partial(
        pl.kernel, mesh=mesh, out_type=out_type, scratch_types=scratch,
        compiler_params=pltpu.CompilerParams(needs_layout_passes=False),
    )(body)


_pass1 = _make_pass(1, NB1)
_pass2 = _make_pass(2, NB2)
_pass3 = _make_pass(3, NB3)


def _make_final():
    mesh = plsc.VectorSubcoreMesh(core_axis_name="c", subcore_axis_name="s")
    SLW = 2 * NB3 * C // NS
    scratch = [
        pltpu.VMEM((4 * NB3 * C // NS,), jnp.float32),
        pltpu.VMEM((SLW,), jnp.float32),
        pltpu.VMEM((4 * C,), jnp.float32),
        pltpu.VMEM((NS * 4 * C,), jnp.float32),
        pltpu.VMEM_SHARED((NS * 4 * C,), jnp.float32),
        pltpu.VMEM((C,), jnp.int32),
        pltpu.VMEM((C,), jnp.float32),
        pltpu.VMEM((C,), jnp.float32),
        pltpu.SemaphoreType.DMA,
        pltpu.SemaphoreType.DMA,
        pltpu.SemaphoreType.DMA,
        pltpu.SemaphoreType.DMA,
    ]

    def body(pc_hbm, ps_hbm, pin_hbm, kin_hbm, ain_hbm, out_hbm,
             ring, mslice, tmpw, tmpall, shst, pvm, kvm, avm,
             sem0, sem1, sem2, sem3):
        cid = lax.axis_index("c")
        sid = lax.axis_index("s")
        wid = sid * NC + cid
        pltpu.sync_copy(pin_hbm, pvm)
        pltpu.sync_copy(kin_hbm, kvm)
        pltpu.sync_copy(ain_hbm, avm)
        ut, rem, acc = _sc_merge(
            NB3, 3, sid, ring, (sem0, sem1, sem2, sem3),
            mslice, tmpw, tmpall, shst, pc_hbm, ps_hbm,
            pvm[...], kvm[...], avm[...])
        t = lax.bitcast_convert_type(ut, jnp.float32)
        res = -((acc + rem * t) / float(K))
        kvm[...] = res

        @pl.when(wid == 0)
        def _w():
            pltpu.sync_copy(kvm, out_hbm)

    return functools.partial(
        pl.kernel, mesh=mesh,
        out_type=jax.ShapeDtypeStruct((C,), jnp.float32),
        scratch_types=scratch,
        compiler_params=pltpu.CompilerParams(needs_layout_passes=False),
    )(body)


_final = _make_final()


def kernel(input):
    xf = input.reshape(-1)
    zi = jnp.zeros((C,), jnp.int32)
    zf = jnp.zeros((C,), jnp.float32)
    c1, s1 = _pass1(xf)
    c2, s2, p1, kk1, a1 = _pass2(xf, c1, s1, zi, zf, zf)
    c3, s3, p2, kk2, a2 = _pass3(xf, c2, s2, p1, kk1, a1)
    return _final(c3, s3, p2, kk2, a2)
